# Initial kernel scaffold; baseline (speedup 1.0000x reference)
import jax
import jax.numpy as jnp
from jax.experimental import pallas as pl


def kernel(x, edge_index, W1, b1, W2, b2, Wfc, bfc):
    raise NotImplementedError("write your pallas kernel here")


# speedup vs baseline: 215.8449x; 215.8449x over previous
import functools

import jax
import jax.numpy as jnp
import numpy as np
from jax import lax
from jax.experimental import pallas as pl
from jax.experimental.pallas import tpu as pltpu
from jax.experimental.pallas import tpu_sc as plsc

N = 10000
E = 320000
D = 128
NC = 2
NS = 16
EPT = E // (NC * NS)
CHUNK = 128
NFULL = EPT // CHUNK
REM = EPT - NFULL * CHUNK
NP8 = 10112
RPT = NP8 // NS
RFULL = RPT // CHUNK
RREM = RPT - RFULL * CHUNK
DEGW = 16

_mesh = plsc.VectorSubcoreMesh(core_axis_name="core", subcore_axis_name="subcore")


def _i32(v):
    return jnp.int32(v)


def _fill(ref, val):
    v = jnp.full((16,), val, jnp.float32)
    for i in range(ref.shape[0]):
        for j in range(0, ref.shape[1], 16):
            ref[i, pl.ds(j, 16)] = v


def _sc_scatter(g, src, dst):
    @functools.partial(
        pl.kernel,
        out_type=jax.ShapeDtypeStruct((NC, NP8, D), jnp.float32),
        mesh=_mesh,
        scratch_types=[
            pltpu.VMEM((CHUNK,), jnp.int32),
            pltpu.VMEM((CHUNK,), jnp.int32),
            pltpu.VMEM((CHUNK, D), jnp.float32),
            pltpu.VMEM((REM,), jnp.int32),
            pltpu.VMEM((REM,), jnp.int32),
            pltpu.VMEM((REM, D), jnp.float32),
            pltpu.VMEM((CHUNK, D), jnp.float32),
            pltpu.VMEM_SHARED((NP8, D), jnp.float32),
        ],
    )
    def k(g_hbm, src_hbm, dst_hbm, out_hbm, idx_s, idx_d, rows, idx_s2,
          idx_d2, rows2, zbuf, acc):
        c = lax.axis_index("core")
        s = lax.axis_index("subcore")
        _fill(zbuf, 0.0)

        rbase = s * _i32(RPT)

        for k0 in range(RFULL):
            pltpu.sync_copy(zbuf, acc.at[pl.ds(rbase + _i32(k0 * CHUNK), CHUNK)])

        pltpu.sync_copy(zbuf.at[pl.ds(0, RREM)],
                        acc.at[pl.ds(rbase + _i32(RFULL * CHUNK), RREM)])
        plsc.subcore_barrier()

        ebase = (c * _i32(NS) + s) * _i32(EPT)

        @pl.loop(_i32(0), _i32(NFULL))
        def _(i):
            off = ebase + i.astype(jnp.int32) * _i32(CHUNK)
            pltpu.sync_copy(src_hbm.at[pl.ds(off, CHUNK)], idx_s)
            pltpu.sync_copy(dst_hbm.at[pl.ds(off, CHUNK)], idx_d)
            pltpu.sync_copy(g_hbm.at[idx_s], rows)
            pltpu.sync_copy(rows, acc.at[idx_d], add=True)

        off2 = ebase + _i32(NFULL * CHUNK)
        pltpu.sync_copy(src_hbm.at[pl.ds(off2, REM)], idx_s2)
        pltpu.sync_copy(dst_hbm.at[pl.ds(off2, REM)], idx_d2)
        pltpu.sync_copy(g_hbm.at[idx_s2], rows2)
        pltpu.sync_copy(rows2, acc.at[idx_d2], add=True)

        plsc.subcore_barrier()

        for k1 in range(RFULL):
            r0 = rbase + _i32(k1 * CHUNK)
            pltpu.sync_copy(acc.at[pl.ds(r0, CHUNK)], rows)
            pltpu.sync_copy(rows, out_hbm.at[c, pl.ds(r0, CHUNK)])

        r1 = rbase + _i32(RFULL * CHUNK)
        pltpu.sync_copy(acc.at[pl.ds(r1, RREM)], rows.at[pl.ds(0, RREM)])
        pltpu.sync_copy(rows.at[pl.ds(0, RREM)], out_hbm.at[c, pl.ds(r1, RREM)])

    return k(g, src, dst)


def _sc_degree(dst):
    @functools.partial(
        pl.kernel,
        out_type=jax.ShapeDtypeStruct((NC, NP8, DEGW), jnp.float32),
        mesh=_mesh,
        scratch_types=[
            pltpu.VMEM((CHUNK,), jnp.int32),
            pltpu.VMEM((REM,), jnp.int32),
            pltpu.VMEM((CHUNK, DEGW), jnp.float32),
            pltpu.VMEM((CHUNK, DEGW), jnp.float32),
            pltpu.VMEM_SHARED((NP8, DEGW), jnp.float32),
        ],
    )
    def k(dst_hbm, out_hbm, idx_d, idx_d2, ones, zbuf, acc):
        c = lax.axis_index("core")
        s = lax.axis_index("subcore")
        _fill(zbuf, 0.0)
        _fill(ones, 1.0)

        rbase = s * _i32(RPT)

        for k0 in range(RFULL):
            pltpu.sync_copy(zbuf, acc.at[pl.ds(rbase + _i32(k0 * CHUNK), CHUNK)])

        pltpu.sync_copy(zbuf.at[pl.ds(0, RREM)],
                        acc.at[pl.ds(rbase + _i32(RFULL * CHUNK), RREM)])
        plsc.subcore_barrier()

        ebase = (c * _i32(NS) + s) * _i32(EPT)

        @pl.loop(_i32(0), _i32(NFULL))
        def _(i):
            off = ebase + i.astype(jnp.int32) * _i32(CHUNK)
            pltpu.sync_copy(dst_hbm.at[pl.ds(off, CHUNK)], idx_d)
            pltpu.sync_copy(ones, acc.at[idx_d], add=True)

        off2 = ebase + _i32(NFULL * CHUNK)
        pltpu.sync_copy(dst_hbm.at[pl.ds(off2, REM)], idx_d2)
        pltpu.sync_copy(ones.at[pl.ds(0, REM)], acc.at[idx_d2], add=True)

        plsc.subcore_barrier()

        for k1 in range(RFULL):
            r0 = rbase + _i32(k1 * CHUNK)
            pltpu.sync_copy(acc.at[pl.ds(r0, CHUNK)], zbuf)
            pltpu.sync_copy(zbuf, out_hbm.at[c, pl.ds(r0, CHUNK)])

        r1 = rbase + _i32(RFULL * CHUNK)
        pltpu.sync_copy(acc.at[pl.ds(r1, RREM)], zbuf.at[pl.ds(0, RREM)])
        pltpu.sync_copy(zbuf.at[pl.ds(0, RREM)], out_hbm.at[c, pl.ds(r1, RREM)])

    return k(dst)


_RB = 1000
_z = np.int32(0)


def _tc_matmul(x, W):
    def body(x_ref, w_ref, h_ref):
        h_ref[...] = jnp.dot(x_ref[...], w_ref[...],
                             preferred_element_type=jnp.float32,
                             precision=lax.Precision.HIGHEST)

    return pl.pallas_call(
        body,
        grid=(N // _RB,),
        in_specs=[
            pl.BlockSpec((_RB, D), lambda i: (i, _z)),
            pl.BlockSpec((D, D), lambda i: (_z, _z)),
        ],
        out_specs=pl.BlockSpec((_RB, D), lambda i: (i, _z)),
        out_shape=jax.ShapeDtypeStruct((N, D), jnp.float32),
    )(x, W)


def _tc_scale(h, dega, degb):
    def body(h_ref, da_ref, db_ref, g_ref, dinv_ref):
        dinv = lax.rsqrt(da_ref[...] + db_ref[...] + 1.0)
        dinv_ref[...] = dinv
        g_ref[...] = h_ref[...] * dinv

    return pl.pallas_call(
        body,
        grid=(N // _RB,),
        in_specs=[
            pl.BlockSpec((_RB, D), lambda i: (i, _z)),
            pl.BlockSpec((_RB, 1), lambda i: (i, _z)),
            pl.BlockSpec((_RB, 1), lambda i: (i, _z)),
        ],
        out_specs=[
            pl.BlockSpec((_RB, D), lambda i: (i, _z)),
            pl.BlockSpec((_RB, 1), lambda i: (i, _z)),
        ],
        out_shape=[
            jax.ShapeDtypeStruct((N, D), jnp.float32),
            jax.ShapeDtypeStruct((N, 1), jnp.float32),
        ],
    )(h, dega, degb)


def _tc_mid(acc2, g, dinv, b, W):
    def body(a_ref, g_ref, dinv_ref, b_ref, w_ref, o_ref):
        u = dinv_ref[...] * (a_ref[0] + a_ref[1] + g_ref[...]) + b_ref[...]
        u = jnp.maximum(u, 0.0)
        o_ref[...] = jnp.dot(u, w_ref[...],
                             preferred_element_type=jnp.float32,
                             precision=lax.Precision.HIGHEST) * dinv_ref[...]

    return pl.pallas_call(
        body,
        grid=(N // _RB,),
        in_specs=[
            pl.BlockSpec((NC, _RB, D), lambda i: (_z, i, _z)),
            pl.BlockSpec((_RB, D), lambda i: (i, _z)),
            pl.BlockSpec((_RB, 1), lambda i: (i, _z)),
            pl.BlockSpec((1, D), lambda i: (_z, _z)),
            pl.BlockSpec((D, D), lambda i: (_z, _z)),
        ],
        out_specs=pl.BlockSpec((_RB, D), lambda i: (i, _z)),
        out_shape=jax.ShapeDtypeStruct((N, D), jnp.float32),
    )(acc2, g, dinv, b, W)


def _tc_post(acc2, g, dinv, b, W, bout):
    def body(a_ref, g_ref, dinv_ref, b_ref, w_ref, bo_ref, o_ref):
        u = dinv_ref[...] * (a_ref[0] + a_ref[1] + g_ref[...]) + b_ref[...]
        u = jnp.maximum(u, 0.0)
        o_ref[...] = jnp.dot(u, w_ref[...],
                             preferred_element_type=jnp.float32,
                             precision=lax.Precision.HIGHEST) + bo_ref[...]

    return pl.pallas_call(
        body,
        grid=(N // _RB,),
        in_specs=[
            pl.BlockSpec((NC, _RB, D), lambda i: (_z, i, _z)),
            pl.BlockSpec((_RB, D), lambda i: (i, _z)),
            pl.BlockSpec((_RB, 1), lambda i: (i, _z)),
            pl.BlockSpec((1, D), lambda i: (_z, _z)),
            pl.BlockSpec((D, D), lambda i: (_z, _z)),
            pl.BlockSpec((1, D), lambda i: (_z, _z)),
        ],
        out_specs=pl.BlockSpec((_RB, D), lambda i: (i, _z)),
        out_shape=jax.ShapeDtypeStruct((N, D), jnp.float32),
    )(acc2, g, dinv, b, W, bout)


def kernel(x, edge_index, W1, b1, W2, b2, Wfc, bfc):
    x = x.astype(jnp.float32)
    src = edge_index[0].astype(jnp.int32)
    dst = edge_index[1].astype(jnp.int32)
    W1 = W1.astype(jnp.float32)
    W2 = W2.astype(jnp.float32)
    Wfc = Wfc.astype(jnp.float32)
    b1 = b1.astype(jnp.float32).reshape(1, D)
    b2 = b2.astype(jnp.float32).reshape(1, D)
    bfc = bfc.astype(jnp.float32).reshape(1, D)

    degp = _sc_degree(dst)
    h1 = _tc_matmul(x, W1)
    g1, dinv = _tc_scale(h1, degp[0, :, 0:1], degp[1, :, 0:1])
    acc1 = _sc_scatter(g1, src, dst)
    g2 = _tc_mid(acc1, g1, dinv, b1, W2)
    acc2 = _sc_scatter(g2, src, dst)
    return _tc_post(acc2, g2, dinv, b2, Wfc, bfc).astype(jnp.float64)
